# baseline (speedup 1.0000x reference)
import functools

import jax
import jax.numpy as jnp
from jax import lax
from jax.experimental import pallas as pl
from jax.experimental.pallas import tpu as pltpu
from jax.experimental.pallas import tpu_sc as plsc

_NUM_NODES = 1000000
_EMBED_DIM = 64
_BATCH = 16384

_INFO = plsc.get_sparse_core_info()
_NC = _INFO.num_cores
_NS = _INFO.num_subcores
_NW = _NC * _NS
_FULL_TCOLS = _NUM_NODES // 128
_TAIL_BASE = _FULL_TCOLS * 128
_CH = 4
_W = _CH * 128
_PER_W = (_FULL_TCOLS + _NW - 1) // _NW
_NCH = (_PER_W + _CH - 1) // _CH
_CAP = 1024
_BCAP = 112
_DUMP = _BATCH
_BIG = 1 << 30


@functools.partial(
    pl.kernel,
    mesh=plsc.VectorSubcoreMesh(core_axis_name="c", subcore_axis_name="s"),
    out_type=jax.ShapeDtypeStruct(((_BATCH + 1) * 128,), jnp.float32),
    scratch_types=[
        pltpu.VMEM((_BATCH,), jnp.int32),
        pltpu.VMEM((_CAP + 16,), jnp.int32),
        pltpu.VMEM((_CAP + 16,), jnp.int32),
        pltpu.VMEM((_BCAP + 16,), jnp.int32),
        pltpu.VMEM((_BCAP + 16,), jnp.int32),
        pltpu.VMEM((_EMBED_DIM, _W), jnp.float32),
        pltpu.VMEM((_EMBED_DIM, _W), jnp.float32),
        pltpu.VMEM((_EMBED_DIM, 64), jnp.float32),
        pltpu.VMEM((16 * 128,), jnp.float32),
        pltpu.SemaphoreType.DMA,
        pltpu.SemaphoreType.DMA,
        pltpu.SemaphoreType.DMA,
    ],
    compiler_params=pltpu.CompilerParams(needs_layout_passes=False),
)
def _r4_kernel(idx_hbm, wt_hbm, out_hbm, idx_v, sel_i, sel_p, bkt_i, bkt_p,
               bufa, bufb, tailb, rowring, sema, semb, semr):
    wid = lax.axis_index("s") * _NC + lax.axis_index("c")
    lo_col = wid * _PER_W
    hi_col = jnp.minimum(lo_col + _PER_W, _FULL_TCOLS)

    pltpu.sync_copy(idx_hbm, idx_v)
    lanes = lax.iota(jnp.int32, 16)
    rows = [lanes + 16 * k for k in range(4)]

    lo_n = lo_col * 128
    hi_n = jnp.where(wid == _NW - 1, _BIG, hi_col * 128)

    def prescan(g, cnt):
        iv = idx_v[pl.ds(g * 16, 16)]
        m = (iv >= lo_n) & (iv < hi_n)
        plsc.store_compressed(sel_i.at[pl.ds(cnt, 16)], iv, mask=m)
        plsc.store_compressed(sel_p.at[pl.ds(cnt, 16)], g * 16 + lanes,
                              mask=m)
        return cnt + plsc.all_reduce_population_count(m)[0]

    n_sel = lax.fori_loop(0, _BATCH // 16, prescan, jnp.int32(0))
    n_sgrp = lax.div(n_sel + 15, jnp.int32(16))

    def chunk_col(t):
        c = lo_col + t * _CH
        return jnp.minimum(c, hi_col - _CH)

    def fetch(t, buf, sem):
        c0 = pl.multiple_of(chunk_col(t) * 128, 128)
        pltpu.make_async_copy(
            wt_hbm.at[:, pl.ds(c0, _W)], buf, sem).start()

    def drain_chunk(buf, sem):
        pltpu.make_async_copy(
            wt_hbm.at[:, pl.ds(0, _W)], buf, sem).wait()

    def bucket(base_n, width):
        def scan(q, cnt):
            si = sel_i[pl.ds(q * 16, 16)]
            sp = sel_p[pl.ds(q * 16, 16)]
            m = (lanes + q * 16 < n_sel) & (si >= base_n) & \
                (si < base_n + width)
            plsc.store_compressed(bkt_i.at[pl.ds(cnt, 16)], si - base_n,
                                  mask=m)
            plsc.store_compressed(bkt_p.at[pl.ds(cnt, 16)], sp, mask=m)
            return cnt + plsc.all_reduce_population_count(m)[0]

        return lax.fori_loop(0, n_sgrp, scan, jnp.int32(0))

    def extract(src_buf, bcnt, fired):
        def group(q, fired):
            @pl.when(fired >= 16)
            def _():
                pltpu.make_async_copy(
                    out_hbm.at[pl.ds(0, 16 * 128)], rowring, semr).wait()
            bi = bkt_i[pl.ds(q * 16, 16)]
            bp = bkt_p[pl.ds(q * 16, 16)]
            valid = lanes + q * 16 < bcnt
            validi = valid.astype(jnp.int32)
            col = jnp.where(valid, bi, 0)
            pos = jnp.where(valid, bp, _DUMP)
            for k in range(16):
                ck = jnp.full((16,), col[k], jnp.int32)

                @pl.when(validi[k] == 1)
                def _():
                    for r in range(4):
                        v = plsc.load_gather(src_buf, [rows[r], ck])
                        rowring[pl.ds(k * 128 + 16 * r, 16)] = v
                pltpu.make_async_copy(
                    rowring.at[pl.ds(k * 128, 128)],
                    out_hbm.at[pl.ds(pos[k] * 128, 128)], semr).start()
            return fired + 16

        n_grp = lax.div(bcnt + 15, jnp.int32(16))
        return lax.fori_loop(0, n_grp, group, fired)

    fetch(0, bufa, sema)

    def stream(h, fired):
        t = h * 2
        fetch(t + 1, bufb, semb)
        b0 = bucket(chunk_col(t) * 128, _W)
        drain_chunk(bufa, sema)
        fired = extract(bufa, b0, fired)
        fetch(t + 2, bufa, sema)
        b1 = bucket(chunk_col(t + 1) * 128, _W)
        drain_chunk(bufb, semb)
        fired = extract(bufb, b1, fired)
        return fired

    fired = lax.fori_loop(0, _NCH // 2, stream, jnp.int32(0))
    drain_chunk(bufa, sema)

    pltpu.sync_copy(wt_hbm.at[:, pl.ds(_TAIL_BASE, 64)], tailb)
    bt = bucket(_TAIL_BASE, 64)
    fired = extract(tailb, bt, fired)

    @pl.when(fired > 0)
    def _():
        pltpu.make_async_copy(
            out_hbm.at[pl.ds(0, 16 * 128)], rowring, semr).wait()


def kernel(indices, weight):
    flat = _r4_kernel(indices.astype(jnp.int32), weight.T)
    return flat.reshape(_BATCH + 1, 128)[:_BATCH, :_EMBED_DIM]


# speedup vs baseline: 4.2720x; 4.2720x over previous
import functools

import jax
import jax.numpy as jnp
from jax import lax
from jax.experimental import pallas as pl
from jax.experimental.pallas import tpu as pltpu
from jax.experimental.pallas import tpu_sc as plsc

_NUM_NODES = 1000000
_EMBED_DIM = 64
_BATCH = 16384

_INFO = plsc.get_sparse_core_info()
_NC = _INFO.num_cores
_NS = _INFO.num_subcores
_NW = _NC * _NS
_B_PER_W = _BATCH // _NW
_NBUF = 8
_NGRP = _B_PER_W // 16


@functools.partial(
    pl.kernel,
    mesh=plsc.VectorSubcoreMesh(core_axis_name="c", subcore_axis_name="s"),
    out_type=jax.ShapeDtypeStruct((_BATCH * _EMBED_DIM,), jnp.float32),
    scratch_types=[
        pltpu.VMEM((_B_PER_W + 16,), jnp.int32),
        pltpu.VMEM((_B_PER_W * _EMBED_DIM,), jnp.float32),
    ]
    + [pltpu.VMEM((_EMBED_DIM, 128), jnp.float32) for _ in range(_NBUF)]
    + [pltpu.SemaphoreType.DMA for _ in range(_NBUF)],
    compiler_params=pltpu.CompilerParams(needs_layout_passes=False),
)
def _gather_kernel(idx_hbm, wt_hbm, out_hbm, idx_v, out_v, *blocks_and_sems):
    blocks = blocks_and_sems[:_NBUF]
    sems = blocks_and_sems[_NBUF:]
    wid = lax.axis_index("s") * _NC + lax.axis_index("c")
    base = wid * _B_PER_W
    pltpu.sync_copy(idx_hbm.at[pl.ds(base, _B_PER_W)],
                    idx_v.at[pl.ds(0, _B_PER_W)])

    rows = [lax.iota(jnp.int32, 16) + 16 * k for k in range(4)]

    def tcol_of(i):
        return pl.multiple_of(
            lax.shift_left(lax.shift_right_logical(i, 7), 7), 128)

    def fire(i, b):
        pltpu.make_async_copy(
            wt_hbm.at[:, pl.ds(tcol_of(i), 128)], blocks[b], sems[b]).start()

    iv0 = idx_v[pl.ds(0, 16)]
    for b in range(_NBUF):
        fire(iv0[b], b)

    def group(g, _):
        jo = g * 16
        iv = idx_v[pl.ds(jo, 16)]
        ivn = idx_v[pl.ds(jo + _NBUF, 16)]
        for k in range(16):
            b = k % _NBUF
            j = jo + k
            pltpu.make_async_copy(
                wt_hbm.at[:, pl.ds(0, 128)], blocks[b], sems[b]).wait()
            lane = jnp.full((16,), iv[k] & 127, jnp.int32)
            for r in range(4):
                v = plsc.load_gather(blocks[b], [rows[r], lane])
                out_v[pl.ds(j * _EMBED_DIM + 16 * r, 16)] = v

            @pl.when(j < _B_PER_W - _NBUF)
            def _():
                fire(ivn[k], b)

        return _

    lax.fori_loop(0, _NGRP, group, None)
    pltpu.sync_copy(out_v, out_hbm.at[pl.ds(base * _EMBED_DIM,
                                            _B_PER_W * _EMBED_DIM)])


def kernel(indices, weight):
    flat = _gather_kernel(indices.astype(jnp.int32), weight.T)
    return flat.reshape(_BATCH, _EMBED_DIM)
